# baseline (speedup 1.0000x reference)
import functools

import jax
import jax.numpy as jnp
from jax.experimental import pallas as pl
from jax.experimental.pallas import tpu as pltpu

_NSTEPS = 8


def _rk4_kernel(x_ref, a_ref, b_ref, o_ref, *, W, G, nsteps):
    CH = a_ref.shape[1]
    L = G * W

    a_cat = jnp.concatenate([a_ref[0], a_ref[1], a_ref[2]], axis=1)
    bias = jnp.broadcast_to(b_ref[...], (CH, L))

    y = jnp.concatenate(
        [x_ref[g].reshape(CH, W) for g in range(G)], axis=1)
    y = y.astype(jnp.float32)

    q = jax.lax.broadcasted_iota(jnp.int32, (1, L), 1)
    wq = q % W
    mask_l = wq != 0
    mask_r = wq != (W - 1)

    zero = jnp.bfloat16(0.0)
    bf16 = jnp.bfloat16

    def odefunc(zb):
        zl = jnp.where(mask_l, pltpu.roll(zb, 1, axis=1), zero)
        zr = jnp.where(mask_r, pltpu.roll(zb, L - 1, axis=1), zero)
        zs = jnp.concatenate([zl, zb, zr], axis=0)
        acc = jnp.dot(a_cat, zs, preferred_element_type=jnp.float32)
        return jnp.tanh(acc + bias).astype(bf16)

    dt = 1.0 / nsteps

    def rk_step(_, yc):
        k1 = odefunc(yc.astype(bf16))
        k2 = odefunc((yc + (0.5 * dt) * k1.astype(jnp.float32)).astype(bf16))
        k3 = odefunc((yc + (0.5 * dt) * k2.astype(jnp.float32)).astype(bf16))
        k4 = odefunc((yc + dt * k3.astype(jnp.float32)).astype(bf16))
        ks = (k1.astype(jnp.float32) + k4.astype(jnp.float32)
              + 2.0 * (k2.astype(jnp.float32) + k3.astype(jnp.float32)))
        return yc + (dt / 6.0) * ks

    y = jax.lax.fori_loop(0, nsteps, rk_step, y)

    y = y.astype(o_ref.dtype)
    C = o_ref.shape[1]
    for g in range(G):
        o_ref[g] = y[:, g * W:(g + 1) * W].reshape(C, CH // C, W)


def kernel(x_nchw, w_oihw, b):
    N, C, H, W = x_nchw.shape
    CH = C * H

    G = max(1, 2048 // W)
    while G > 1 and (N % G != 0 or N // G < 2):
        G //= 2
    B = N // G

    bands = jnp.stack([jnp.eye(H, k=-1, dtype=w_oihw.dtype),
                       jnp.eye(H, k=0, dtype=w_oihw.dtype),
                       jnp.eye(H, k=1, dtype=w_oihw.dtype)])
    a_all = jnp.einsum('oidw,dhk->wohik', w_oihw, bands).reshape(3, CH, CH)
    a_all = a_all.astype(jnp.bfloat16)
    b_col = jnp.repeat(b, H).reshape(CH, 1)

    fn = functools.partial(_rk4_kernel, W=W, G=G, nsteps=_NSTEPS)
    out = pl.pallas_call(
        fn,
        out_shape=jax.ShapeDtypeStruct((N, C, H, W), x_nchw.dtype),
        grid=(B,),
        in_specs=[
            pl.BlockSpec((G, C, H, W), lambda n: (n, 0, 0, 0)),
            pl.BlockSpec((3, CH, CH), lambda n: (0, 0, 0)),
            pl.BlockSpec((CH, 1), lambda n: (0, 0)),
        ],
        out_specs=pl.BlockSpec((G, C, H, W), lambda n: (n, 0, 0, 0)),
        compiler_params=pltpu.CompilerParams(
            dimension_semantics=("arbitrary",)),
    )(x_nchw, a_all, b_col)

    return out


# speedup vs baseline: 1.0116x; 1.0116x over previous
import functools

import jax
import jax.numpy as jnp
from jax.experimental import pallas as pl
from jax.experimental.pallas import tpu as pltpu

_NSTEPS = 8


def _rk4_kernel(x_ref, a_ref, b_ref, o_ref, *, W, G, nchain, nsteps):
    CH = a_ref.shape[1]
    L = G * W
    C = o_ref.shape[1]

    a_cat = jnp.concatenate([a_ref[0], a_ref[1], a_ref[2]], axis=1)
    a_cat = a_cat.astype(jnp.bfloat16)
    bias = jnp.broadcast_to(b_ref[...], (CH, L))

    ys = tuple(
        jnp.concatenate(
            [x_ref[j * G + g].reshape(CH, W) for g in range(G)],
            axis=1).astype(jnp.float32)
        for j in range(nchain))

    q = jax.lax.broadcasted_iota(jnp.int32, (1, L), 1)
    wq = q % W
    mask_l = wq != 0
    mask_r = wq != (W - 1)

    zero = jnp.bfloat16(0.0)

    def odefunc(z):
        zb = z.astype(jnp.bfloat16)
        zl = jnp.where(mask_l, pltpu.roll(zb, 1, axis=1), zero)
        zr = jnp.where(mask_r, pltpu.roll(zb, L - 1, axis=1), zero)
        zs = jnp.concatenate([zl, zb, zr], axis=0)
        acc = jnp.dot(a_cat, zs, preferred_element_type=jnp.float32)
        return jnp.tanh(acc + bias)

    dt = 1.0 / nsteps

    def rk_step(_, carry):
        k1 = tuple(odefunc(y) for y in carry)
        k2 = tuple(odefunc(y + (0.5 * dt) * k) for y, k in zip(carry, k1))
        k3 = tuple(odefunc(y + (0.5 * dt) * k) for y, k in zip(carry, k2))
        k4 = tuple(odefunc(y + dt * k) for y, k in zip(carry, k3))
        return tuple(
            y + (dt / 6.0) * (a + 2.0 * (b2 + b3) + b4)
            for y, a, b2, b3, b4 in zip(carry, k1, k2, k3, k4))

    ys = jax.lax.fori_loop(0, nsteps, rk_step, ys)

    for j in range(nchain):
        y = ys[j].astype(o_ref.dtype)
        for g in range(G):
            o_ref[j * G + g] = y[:, g * W:(g + 1) * W].reshape(C, CH // C, W)


def kernel(x_nchw, w_oihw, b):
    N, C, H, W = x_nchw.shape
    CH = C * H

    G = max(1, 2048 // W)
    while G > 1 and N % G != 0:
        G //= 2
    nchain = 2 if N % (2 * G) == 0 and N // (2 * G) >= 1 else 1
    B = N // (nchain * G)

    bands = jnp.stack([jnp.eye(H, k=-1, dtype=w_oihw.dtype),
                       jnp.eye(H, k=0, dtype=w_oihw.dtype),
                       jnp.eye(H, k=1, dtype=w_oihw.dtype)])
    a_all = jnp.einsum('oidw,dhk->wohik', w_oihw, bands).reshape(3, CH, CH)
    b_col = jnp.repeat(b, H).reshape(CH, 1)

    fn = functools.partial(_rk4_kernel, W=W, G=G, nchain=nchain,
                           nsteps=_NSTEPS)
    out = pl.pallas_call(
        fn,
        out_shape=jax.ShapeDtypeStruct((N, C, H, W), x_nchw.dtype),
        grid=(B,),
        in_specs=[
            pl.BlockSpec((nchain * G, C, H, W), lambda n: (n, 0, 0, 0)),
            pl.BlockSpec((3, CH, CH), lambda n: (0, 0, 0)),
            pl.BlockSpec((CH, 1), lambda n: (0, 0)),
        ],
        out_specs=pl.BlockSpec((nchain * G, C, H, W), lambda n: (n, 0, 0, 0)),
        compiler_params=pltpu.CompilerParams(
            dimension_semantics=("arbitrary",)),
    )(x_nchw, a_all, b_col)

    return out


# speedup vs baseline: 1.0567x; 1.0447x over previous
import functools

import jax
import jax.numpy as jnp
from jax.experimental import pallas as pl
from jax.experimental.pallas import tpu as pltpu

_NSTEPS = 8


def _rk4_kernel(x_ref, a_ref, b_ref, o_ref, *, W, G, nchain, nsteps):
    CH = a_ref.shape[1]
    L = G * W
    C = o_ref.shape[1]

    a_cat = jnp.concatenate([a_ref[0], a_ref[1], a_ref[2]], axis=1)
    a_cat = a_cat.astype(jnp.bfloat16)
    bias = jnp.broadcast_to(b_ref[...], (CH, L))

    ys = tuple(
        jnp.concatenate(
            [x_ref[j * G + g].reshape(CH, W) for g in range(G)],
            axis=1).astype(jnp.bfloat16)
        for j in range(nchain))

    q = jax.lax.broadcasted_iota(jnp.int32, (1, L), 1)
    wq = q % W
    mask_l = wq != 0
    mask_r = wq != (W - 1)

    zero = jnp.bfloat16(0.0)

    def odefunc(zb):
        zl = jnp.where(mask_l, pltpu.roll(zb, 1, axis=1), zero)
        zr = jnp.where(mask_r, pltpu.roll(zb, L - 1, axis=1), zero)
        zs = jnp.concatenate([zl, zb, zr], axis=0)
        acc = jnp.dot(a_cat, zs, preferred_element_type=jnp.float32)
        return jnp.tanh(acc + bias)

    dt = 1.0 / nsteps

    bf16 = jnp.bfloat16

    def rk_step(_, carry):
        k1 = tuple(odefunc(y) for y in carry)
        k2 = tuple(odefunc((y + (0.5 * dt) * k).astype(bf16))
                   for y, k in zip(carry, k1))
        k3 = tuple(odefunc((y + (0.5 * dt) * k).astype(bf16))
                   for y, k in zip(carry, k2))
        k4 = tuple(odefunc((y + dt * k).astype(bf16))
                   for y, k in zip(carry, k3))
        return tuple(
            (y + (dt / 6.0) * (a + 2.0 * (b2 + b3) + b4)).astype(bf16)
            for y, a, b2, b3, b4 in zip(carry, k1, k2, k3, k4))

    ys = jax.lax.fori_loop(0, nsteps, rk_step, ys)

    for j in range(nchain):
        y = ys[j].astype(o_ref.dtype)
        for g in range(G):
            o_ref[j * G + g] = y[:, g * W:(g + 1) * W].reshape(C, CH // C, W)


def kernel(x_nchw, w_oihw, b):
    N, C, H, W = x_nchw.shape
    CH = C * H

    G = max(1, 2048 // W)
    while G > 1 and N % G != 0:
        G //= 2
    nchain = 2 if N % (2 * G) == 0 and N // (2 * G) >= 1 else 1
    B = N // (nchain * G)

    bands = jnp.stack([jnp.eye(H, k=-1, dtype=w_oihw.dtype),
                       jnp.eye(H, k=0, dtype=w_oihw.dtype),
                       jnp.eye(H, k=1, dtype=w_oihw.dtype)])
    a_all = jnp.einsum('oidw,dhk->wohik', w_oihw, bands).reshape(3, CH, CH)
    b_col = jnp.repeat(b, H).reshape(CH, 1)

    fn = functools.partial(_rk4_kernel, W=W, G=G, nchain=nchain,
                           nsteps=_NSTEPS)
    out = pl.pallas_call(
        fn,
        out_shape=jax.ShapeDtypeStruct((N, C, H, W), x_nchw.dtype),
        grid=(B,),
        in_specs=[
            pl.BlockSpec((nchain * G, C, H, W), lambda n: (n, 0, 0, 0)),
            pl.BlockSpec((3, CH, CH), lambda n: (0, 0, 0)),
            pl.BlockSpec((CH, 1), lambda n: (0, 0)),
        ],
        out_specs=pl.BlockSpec((nchain * G, C, H, W), lambda n: (n, 0, 0, 0)),
        compiler_params=pltpu.CompilerParams(
            dimension_semantics=("arbitrary",)),
    )(x_nchw, a_all, b_col)

    return out


# speedup vs baseline: 1.0958x; 1.0369x over previous
import functools

import jax
import jax.numpy as jnp
from jax.experimental import pallas as pl
from jax.experimental.pallas import tpu as pltpu

_NSTEPS = 8


def _rk4_kernel(x_ref, a_ref, b_ref, o_ref, *scratch, W, G, nchain, nsteps):
    CH = a_ref.shape[1]
    L = G * W
    C = o_ref.shape[1]

    a_cat = jnp.concatenate([a_ref[0], a_ref[1], a_ref[2]], axis=1)
    a_cat = a_cat.astype(jnp.bfloat16)
    bias = jnp.broadcast_to(b_ref[...], (CH, L))

    for j in range(nchain):
        scratch[j][...] = jnp.concatenate(
            [x_ref[j * G + g].reshape(CH, W) for g in range(G)],
            axis=1).astype(jnp.float32)

    q = jax.lax.broadcasted_iota(jnp.int32, (1, L), 1)
    wq = q % W
    mask_l = wq != 0
    mask_r = wq != (W - 1)

    zero = jnp.bfloat16(0.0)

    def odefunc(z):
        zb = z.astype(jnp.bfloat16)
        zl = jnp.where(mask_l, pltpu.roll(zb, 1, axis=1), zero)
        zr = jnp.where(mask_r, pltpu.roll(zb, L - 1, axis=1), zero)
        zs = jnp.concatenate([zl, zb, zr], axis=0)
        acc = jnp.dot(a_cat, zs, preferred_element_type=jnp.float32)
        return jnp.tanh(acc + bias)

    dt = 1.0 / nsteps

    def rk_step(_, carry):
        ys = tuple(s[...] for s in scratch)
        k1 = tuple(odefunc(y) for y in ys)
        k2 = tuple(odefunc(y + (0.5 * dt) * k) for y, k in zip(ys, k1))
        k3 = tuple(odefunc(y + (0.5 * dt) * k) for y, k in zip(ys, k2))
        k4 = tuple(odefunc(y + dt * k) for y, k in zip(ys, k3))
        for j in range(nchain):
            scratch[j][...] = (ys[j] + (dt / 6.0) *
                               (k1[j] + 2.0 * (k2[j] + k3[j]) + k4[j]))
        return carry

    jax.lax.fori_loop(0, nsteps, rk_step, 0)

    for j in range(nchain):
        y = scratch[j][...].astype(o_ref.dtype)
        for g in range(G):
            o_ref[j * G + g] = y[:, g * W:(g + 1) * W].reshape(C, CH // C, W)


def kernel(x_nchw, w_oihw, b):
    N, C, H, W = x_nchw.shape
    CH = C * H

    G = max(1, 2048 // W)
    while G > 1 and N % G != 0:
        G //= 2
    nchain = 2 if N % (2 * G) == 0 and N // (2 * G) >= 1 else 1
    B = N // (nchain * G)

    bands = jnp.stack([jnp.eye(H, k=-1, dtype=w_oihw.dtype),
                       jnp.eye(H, k=0, dtype=w_oihw.dtype),
                       jnp.eye(H, k=1, dtype=w_oihw.dtype)])
    a_all = jnp.einsum('oidw,dhk->wohik', w_oihw, bands).reshape(3, CH, CH)
    b_col = jnp.repeat(b, H).reshape(CH, 1)

    fn = functools.partial(_rk4_kernel, W=W, G=G, nchain=nchain,
                           nsteps=_NSTEPS)
    out = pl.pallas_call(
        fn,
        out_shape=jax.ShapeDtypeStruct((N, C, H, W), x_nchw.dtype),
        grid=(B,),
        in_specs=[
            pl.BlockSpec((nchain * G, C, H, W), lambda n: (n, 0, 0, 0)),
            pl.BlockSpec((3, CH, CH), lambda n: (0, 0, 0)),
            pl.BlockSpec((CH, 1), lambda n: (0, 0)),
        ],
        out_specs=pl.BlockSpec((nchain * G, C, H, W), lambda n: (n, 0, 0, 0)),
        scratch_shapes=[pltpu.VMEM((CH, G * W), jnp.float32)
                        for _ in range(nchain)],
        compiler_params=pltpu.CompilerParams(
            dimension_semantics=("arbitrary",)),
    )(x_nchw, a_all, b_col)

    return out
